# baseline (speedup 1.0000x reference)
import jax
import jax.numpy as jnp
from jax import lax
from jax.experimental import pallas as pl
from jax.experimental.pallas import tpu as pltpu

_B = 16384
_C = 1000
_BR = 1024
_NCHUNK = _B // _BR
_NBUF = 8


def _onehot_body(ids_ref, o_ref, *scratch):
    bufs = scratch[:_NBUF]
    sems = scratch[_NBUF:]
    col = lax.broadcasted_iota(jnp.int32, (_BR, _C), 1)
    inflight = [None] * _NBUF
    for k in range(_NCHUNK):
        b = k % _NBUF
        if inflight[b] is not None:
            inflight[b].wait()
        ids = ids_ref[pl.ds(k * _BR, _BR), :]
        in_vocab = (ids >= 0) & (ids < _C)
        mapped = jnp.where(in_vocab, ids, _C - 1)
        bufs[b][...] = jnp.where(col == mapped, 1.0, 0.0).astype(jnp.float32)
        cp = pltpu.make_async_copy(
            bufs[b], o_ref.at[pl.ds(k * _BR, _BR), :], sems[b]
        )
        cp.start()
        inflight[b] = cp
    for b in range(_NBUF):
        if inflight[b] is not None:
            inflight[b].wait()


def kernel(user_ids):
    ids = user_ids.astype(jnp.int32).reshape(_B, 1)
    out = pl.pallas_call(
        _onehot_body,
        in_specs=[pl.BlockSpec(memory_space=pltpu.MemorySpace.VMEM)],
        out_specs=pl.BlockSpec(memory_space=pltpu.MemorySpace.HBM),
        out_shape=jax.ShapeDtypeStruct((_B, _C), jnp.float32),
        scratch_shapes=(
            [pltpu.VMEM((_BR, _C), jnp.float32) for _ in range(_NBUF)]
            + [pltpu.SemaphoreType.DMA for _ in range(_NBUF)]
        ),
    )(ids)
    return out


# speedup vs baseline: 4.1333x; 4.1333x over previous
import jax
import jax.numpy as jnp
from jax import lax
from jax.experimental import pallas as pl
from jax.experimental.pallas import tpu as pltpu

_B = 16384
_C = 1000
_BC = 2048
_GRID = _B // _BC


def _onehot_block(ids_ref, o_ref):
    ids = ids_ref[0]
    in_vocab = (ids >= 0) & (ids < _C)
    mapped = jnp.where(in_vocab, ids, _C - 1)
    row = lax.broadcasted_iota(jnp.int32, (_C, _BC), 0)
    o_ref[...] = jnp.where(row == mapped, 1.0, 0.0).astype(jnp.float32)


def kernel(user_ids):
    ids = user_ids.astype(jnp.int32).reshape(_GRID, 1, _BC)
    out_t = pl.pallas_call(
        _onehot_block,
        grid=(_GRID,),
        in_specs=[pl.BlockSpec((1, 1, _BC), lambda j: (j, 0, 0))],
        out_specs=pl.BlockSpec((_C, _BC), lambda j: (0, j)),
        out_shape=jax.ShapeDtypeStruct((_C, _B), jnp.float32),
    )(ids)
    return out_t.T
